# baseline (speedup 1.0000x reference)
import functools

import jax
import jax.numpy as jnp
from jax import lax
from jax.experimental import pallas as pl
from jax.experimental.pallas import tpu as pltpu
from jax.experimental.pallas import tpu_sc as plsc

N_NODES = 100000
D = 128
BATCH = 50000
S = 10
L = 16

NC = 2
NS = 16
NW = NC * NS

B_PER_W = 1600
BP = NW * B_PER_W
C = 16
NCH = B_PER_W // C
NBUF = 4
NPAIR = NCH // NBUF
NSPLIT = (80, 80)


def _sc_encoder(feat_table, nodes_pad, neigh_flat):
    mesh = plsc.VectorSubcoreMesh(core_axis_name="c", subcore_axis_name="s")

    @functools.partial(
        pl.kernel,
        mesh=mesh,
        out_type=jax.ShapeDtypeStruct((BATCH, 2 * D), jnp.float32),
        scratch_types=[
            pltpu.VMEM((B_PER_W,), jnp.int32),
            pltpu.VMEM((B_PER_W * S,), jnp.int32),
            pltpu.VMEM((NBUF * C * S, D), jnp.float32),
            pltpu.VMEM((NBUF * C, 2 * D), jnp.float32),
        ] + [pltpu.SemaphoreType.DMA] * (2 * NBUF),
    )
    def body(table_h, nodes_h, neigh_h, out_h, sidx, nidx, nrows, obuf, *sems):
        gsem = sems[:NBUF]
        osem = sems[NBUF:]
        wid = lax.axis_index("s") * NC + lax.axis_index("c")
        base = jnp.minimum(wid * B_PER_W, BATCH - B_PER_W)
        pltpu.sync_copy(nodes_h.at[pl.ds(base, B_PER_W)], sidx)
        pltpu.sync_copy(neigh_h.at[pl.ds(base * S, B_PER_W * S)], nidx)

        def issue_gather(ci, b):
            off = ci * C
            pltpu.async_copy(
                table_h.at[sidx.at[pl.ds(off, C)]],
                obuf.at[pl.ds(b * C, C), pl.ds(0, D)],
                gsem[b],
            )
            pos = 0
            for sz in NSPLIT:
                pltpu.async_copy(
                    table_h.at[nidx.at[pl.ds(off * S + pos, sz)]],
                    nrows.at[pl.ds(b * C * S + pos, sz)],
                    gsem[b],
                )
                pos += sz

        def drain_gather(b):
            pltpu.make_async_copy(
                table_h.at[pl.ds(0, C)],
                obuf.at[pl.ds(b * C, C), pl.ds(0, D)],
                gsem[b],
            ).wait()
            pltpu.make_async_copy(
                table_h.at[pl.ds(0, C * S)],
                nrows.at[pl.ds(b * C * S, C * S)],
                gsem[b],
            ).wait()

        def compute(b):
            nb = b * C * S
            ob = b * C

            def row(r, carry):
                rb = nb + r * S
                for g in range(D // L):
                    sl = pl.ds(g * L, L)
                    acc = nrows[rb, sl]
                    for j in range(1, S):
                        acc = acc + nrows[rb + j, sl]
                    obuf[ob + r, pl.ds(D + g * L, L)] = acc * jnp.float32(1.0 / S)
                return carry

            lax.fori_loop(0, C, row, 0, unroll=False)

        def issue_out(ci, b):
            pltpu.async_copy(
                obuf.at[pl.ds(b * C, C)],
                out_h.at[pl.ds(base + ci * C, C)],
                osem[b],
            )

        def drain_out(b):
            pltpu.make_async_copy(
                obuf.at[pl.ds(b * C, C)],
                out_h.at[pl.ds(0, C)],
                osem[b],
            ).wait()

        for b in range(NBUF):
            issue_gather(b, b)
        for b in range(NBUF):
            drain_gather(b)
            compute(b)
            issue_out(b, b)
            issue_gather(b + NBUF, b)

        def steady(gi, carry):
            for b in range(NBUF):
                ci = gi * NBUF + b
                drain_gather(b)
                drain_out(b)
                compute(b)
                issue_out(ci, b)
                issue_gather(ci + NBUF, b)
            return carry

        lax.fori_loop(1, NPAIR - 1, steady, 0, unroll=False)

        for b in range(NBUF):
            ci = (NPAIR - 1) * NBUF + b
            drain_gather(b)
            drain_out(b)
            compute(b)
            issue_out(ci, b)
        for b in range(NBUF):
            drain_out(b)

    return body(feat_table, nodes_pad, neigh_flat)


@jax.jit
def kernel(feat_table, nodes, neigh_idx):
    return _sc_encoder(feat_table, nodes, neigh_idx.reshape(-1))


# speedup vs baseline: 1.4206x; 1.4206x over previous
import functools

import jax
import jax.numpy as jnp
from jax import lax
from jax.experimental import pallas as pl
from jax.experimental.pallas import tpu as pltpu
from jax.experimental.pallas import tpu_sc as plsc

N_NODES = 100000
D = 128
BATCH = 50000
S = 10
L = 16

NC = 2
NS = 16
NW = NC * NS

B_PER_W = 1600
BP = NW * B_PER_W
C = 32
NCH = B_PER_W // C
NBUF = 2
NPAIR = NCH // NBUF
NSPLIT = (112, 112, 96)


def _sc_encoder(feat_table, nodes_pad, neigh_flat):
    mesh = plsc.VectorSubcoreMesh(core_axis_name="c", subcore_axis_name="s")

    @functools.partial(
        pl.kernel,
        mesh=mesh,
        out_type=jax.ShapeDtypeStruct((BATCH, 2 * D), jnp.float32),
        scratch_types=[
            pltpu.VMEM((B_PER_W,), jnp.int32),
            pltpu.VMEM((B_PER_W * S,), jnp.int32),
            pltpu.VMEM((NBUF * C * S, D), jnp.float32),
            pltpu.VMEM((NBUF * C, 2 * D), jnp.float32),
            pltpu.SemaphoreType.DMA,
            pltpu.SemaphoreType.DMA,
            pltpu.SemaphoreType.DMA,
            pltpu.SemaphoreType.DMA,
        ],
    )
    def body(table_h, nodes_h, neigh_h, out_h, sidx, nidx, nrows, obuf,
             gsem0, gsem1, osem0, osem1):
        gsem = (gsem0, gsem1)
        osem = (osem0, osem1)
        wid = lax.axis_index("s") * NC + lax.axis_index("c")
        base = jnp.minimum(wid * B_PER_W, BATCH - B_PER_W)
        pltpu.sync_copy(nodes_h.at[pl.ds(base, B_PER_W)], sidx)
        pltpu.sync_copy(neigh_h.at[pl.ds(base * S, B_PER_W * S)], nidx)

        def issue_gather(ci, b):
            off = ci * C
            pltpu.async_copy(
                table_h.at[sidx.at[pl.ds(off, C)]],
                obuf.at[pl.ds(b * C, C), pl.ds(0, D)],
                gsem[b],
            )
            pos = 0
            for sz in NSPLIT:
                pltpu.async_copy(
                    table_h.at[nidx.at[pl.ds(off * S + pos, sz)]],
                    nrows.at[pl.ds(b * C * S + pos, sz)],
                    gsem[b],
                )
                pos += sz

        def drain_gather(b):
            pltpu.make_async_copy(
                table_h.at[pl.ds(0, C)],
                obuf.at[pl.ds(b * C, C), pl.ds(0, D)],
                gsem[b],
            ).wait()
            pltpu.make_async_copy(
                table_h.at[pl.ds(0, C * S)],
                nrows.at[pl.ds(b * C * S, C * S)],
                gsem[b],
            ).wait()

        def compute(b):
            nb = b * C * S
            ob = b * C

            @plsc.parallel_loop(0, C, step=1, unroll=2)
            def row(r):
                rb = nb + r * S
                for g in range(D // L):
                    sl = pl.ds(g * L, L)
                    acc = nrows[rb, sl]
                    for j in range(1, S):
                        acc = acc + nrows[rb + j, sl]
                    obuf[ob + r, pl.ds(D + g * L, L)] = acc * jnp.float32(1.0 / S)

        def issue_out(ci, b):
            pltpu.async_copy(
                obuf.at[pl.ds(b * C, C)],
                out_h.at[pl.ds(base + ci * C, C)],
                osem[b],
            )

        def drain_out(b):
            pltpu.make_async_copy(
                obuf.at[pl.ds(b * C, C)],
                out_h.at[pl.ds(0, C)],
                osem[b],
            ).wait()

        for b in range(NBUF):
            issue_gather(b, b)
        for b in range(NBUF):
            drain_gather(b)
            compute(b)
            issue_out(b, b)
            issue_gather(b + NBUF, b)

        def steady(gi, carry):
            for b in range(NBUF):
                ci = gi * NBUF + b
                drain_gather(b)
                drain_out(b)
                compute(b)
                issue_out(ci, b)
                issue_gather(ci + NBUF, b)
            return carry

        lax.fori_loop(1, NPAIR - 1, steady, 0, unroll=False)

        for b in range(NBUF):
            ci = (NPAIR - 1) * NBUF + b
            drain_gather(b)
            drain_out(b)
            compute(b)
            issue_out(ci, b)
        for b in range(NBUF):
            drain_out(b)

    return body(feat_table, nodes_pad, neigh_flat)


@jax.jit
def kernel(feat_table, nodes, neigh_idx):
    return _sc_encoder(feat_table, nodes, neigh_idx.reshape(-1))


# speedup vs baseline: 1.4508x; 1.0213x over previous
import functools

import jax
import jax.numpy as jnp
from jax import lax
from jax.experimental import pallas as pl
from jax.experimental.pallas import tpu as pltpu
from jax.experimental.pallas import tpu_sc as plsc

N_NODES = 100000
D = 128
BATCH = 50000
S = 10
L = 16

NC = 2
NS = 16
NW = NC * NS

B_PER_W = 1600
BP = NW * B_PER_W
C = 32
NCH = B_PER_W // C
NBUF = 2
NPAIR = NCH // NBUF
NSPLIT = (128, 128, 64)


def _sc_encoder(feat_table, nodes_pad, neigh_flat):
    mesh = plsc.VectorSubcoreMesh(core_axis_name="c", subcore_axis_name="s")

    @functools.partial(
        pl.kernel,
        mesh=mesh,
        out_type=jax.ShapeDtypeStruct((BATCH, 2 * D), jnp.float32),
        scratch_types=[
            pltpu.VMEM((B_PER_W,), jnp.int32),
            pltpu.VMEM((B_PER_W * S,), jnp.int32),
            pltpu.VMEM((NBUF * C * S, D), jnp.float32),
            pltpu.VMEM((NBUF * C, 2 * D), jnp.float32),
            pltpu.SemaphoreType.DMA,
            pltpu.SemaphoreType.DMA,
            pltpu.SemaphoreType.DMA,
            pltpu.SemaphoreType.DMA,
        ],
    )
    def body(table_h, nodes_h, neigh_h, out_h, sidx, nidx, nrows, obuf,
             gsem0, gsem1, osem0, osem1):
        gsem = (gsem0, gsem1)
        osem = (osem0, osem1)
        wid = lax.axis_index("s") * NC + lax.axis_index("c")
        base = jnp.minimum(wid * B_PER_W, BATCH - B_PER_W)
        pltpu.sync_copy(nodes_h.at[pl.ds(base, B_PER_W)], sidx)
        pltpu.sync_copy(neigh_h.at[pl.ds(base * S, B_PER_W * S)], nidx)

        def issue_gather(ci, b):
            off = ci * C
            pltpu.async_copy(
                table_h.at[sidx.at[pl.ds(off, C)]],
                obuf.at[pl.ds(b * C, C), pl.ds(0, D)],
                gsem[b],
            )
            pos = 0
            for sz in NSPLIT:
                pltpu.async_copy(
                    table_h.at[nidx.at[pl.ds(off * S + pos, sz)]],
                    nrows.at[pl.ds(b * C * S + pos, sz)],
                    gsem[b],
                )
                pos += sz

        def drain_gather(b):
            pltpu.make_async_copy(
                table_h.at[pl.ds(0, C)],
                obuf.at[pl.ds(b * C, C), pl.ds(0, D)],
                gsem[b],
            ).wait()
            pltpu.make_async_copy(
                table_h.at[pl.ds(0, C * S)],
                nrows.at[pl.ds(b * C * S, C * S)],
                gsem[b],
            ).wait()

        def compute(b):
            nb = b * C * S
            ob = b * C

            @plsc.parallel_loop(0, C, step=1, unroll=4)
            def row(r):
                rb = nb + r * S
                for g in range(D // L):
                    sl = pl.ds(g * L, L)
                    acc = nrows[rb, sl]
                    for j in range(1, S):
                        acc = acc + nrows[rb + j, sl]
                    obuf[ob + r, pl.ds(D + g * L, L)] = acc * jnp.float32(1.0 / S)

        def issue_out(ci, b):
            pltpu.async_copy(
                obuf.at[pl.ds(b * C, C)],
                out_h.at[pl.ds(base + ci * C, C)],
                osem[b],
            )

        def drain_out(b):
            pltpu.make_async_copy(
                obuf.at[pl.ds(b * C, C)],
                out_h.at[pl.ds(0, C)],
                osem[b],
            ).wait()

        for b in range(NBUF):
            issue_gather(b, b)
        for b in range(NBUF):
            drain_gather(b)
            compute(b)
            issue_out(b, b)
            issue_gather(b + NBUF, b)

        def steady(gi, carry):
            for b in range(NBUF):
                ci = gi * NBUF + b
                drain_gather(b)
                drain_out(b)
                compute(b)
                issue_out(ci, b)
                issue_gather(ci + NBUF, b)
            return carry

        lax.fori_loop(1, NPAIR - 1, steady, 0, unroll=False)

        for b in range(NBUF):
            ci = (NPAIR - 1) * NBUF + b
            drain_gather(b)
            drain_out(b)
            compute(b)
            issue_out(ci, b)
        for b in range(NBUF):
            drain_out(b)

    return body(feat_table, nodes_pad, neigh_flat)


@jax.jit
def kernel(feat_table, nodes, neigh_idx):
    return _sc_encoder(feat_table, nodes, neigh_idx.reshape(-1))


# speedup vs baseline: 1.4538x; 1.0021x over previous
import functools

import jax
import jax.numpy as jnp
from jax import lax
from jax.experimental import pallas as pl
from jax.experimental.pallas import tpu as pltpu
from jax.experimental.pallas import tpu_sc as plsc

N_NODES = 100000
D = 128
BATCH = 50000
S = 10
L = 16

NC = 2
NS = 16
NW = NC * NS

B_PER_W = 1600
BP = NW * B_PER_W
C = 32
NCH = B_PER_W // C
NBUF = 2
NPAIR = NCH // NBUF
NSPLIT = (320,)


def _sc_encoder(feat_table, nodes_pad, neigh_flat):
    mesh = plsc.VectorSubcoreMesh(core_axis_name="c", subcore_axis_name="s")

    @functools.partial(
        pl.kernel,
        mesh=mesh,
        out_type=jax.ShapeDtypeStruct((BATCH, 2 * D), jnp.float32),
        scratch_types=[
            pltpu.VMEM((B_PER_W,), jnp.int32),
            pltpu.VMEM((B_PER_W * S,), jnp.int32),
            pltpu.VMEM((NBUF * C * S, D), jnp.float32),
            pltpu.VMEM((NBUF * C, 2 * D), jnp.float32),
            pltpu.SemaphoreType.DMA,
            pltpu.SemaphoreType.DMA,
            pltpu.SemaphoreType.DMA,
            pltpu.SemaphoreType.DMA,
        ],
    )
    def body(table_h, nodes_h, neigh_h, out_h, sidx, nidx, nrows, obuf,
             gsem0, gsem1, osem0, osem1):
        gsem = (gsem0, gsem1)
        osem = (osem0, osem1)
        wid = lax.axis_index("s") * NC + lax.axis_index("c")
        base = jnp.minimum(wid * B_PER_W, BATCH - B_PER_W)
        pltpu.sync_copy(nodes_h.at[pl.ds(base, B_PER_W)], sidx)
        pltpu.sync_copy(neigh_h.at[pl.ds(base * S, B_PER_W * S)], nidx)

        def issue_gather(ci, b):
            off = ci * C
            pltpu.async_copy(
                table_h.at[sidx.at[pl.ds(off, C)]],
                obuf.at[pl.ds(b * C, C), pl.ds(0, D)],
                gsem[b],
            )
            pos = 0
            for sz in NSPLIT:
                pltpu.async_copy(
                    table_h.at[nidx.at[pl.ds(off * S + pos, sz)]],
                    nrows.at[pl.ds(b * C * S + pos, sz)],
                    gsem[b],
                )
                pos += sz

        def drain_gather(b):
            pltpu.make_async_copy(
                table_h.at[pl.ds(0, C)],
                obuf.at[pl.ds(b * C, C), pl.ds(0, D)],
                gsem[b],
            ).wait()
            pltpu.make_async_copy(
                table_h.at[pl.ds(0, C * S)],
                nrows.at[pl.ds(b * C * S, C * S)],
                gsem[b],
            ).wait()

        def compute(b):
            nb = b * C * S
            ob = b * C

            @plsc.parallel_loop(0, C, step=1, unroll=4)
            def row(r):
                rb = nb + r * S
                for g in range(D // L):
                    sl = pl.ds(g * L, L)
                    acc = nrows[rb, sl]
                    for j in range(1, S):
                        acc = acc + nrows[rb + j, sl]
                    obuf[ob + r, pl.ds(D + g * L, L)] = acc * jnp.float32(1.0 / S)

        def issue_out(ci, b):
            pltpu.async_copy(
                obuf.at[pl.ds(b * C, C)],
                out_h.at[pl.ds(base + ci * C, C)],
                osem[b],
            )

        def drain_out(b):
            pltpu.make_async_copy(
                obuf.at[pl.ds(b * C, C)],
                out_h.at[pl.ds(0, C)],
                osem[b],
            ).wait()

        for b in range(NBUF):
            issue_gather(b, b)
        for b in range(NBUF):
            drain_gather(b)
            compute(b)
            issue_out(b, b)
            issue_gather(b + NBUF, b)

        def steady(gi, carry):
            for b in range(NBUF):
                ci = gi * NBUF + b
                drain_gather(b)
                drain_out(b)
                compute(b)
                issue_out(ci, b)
                issue_gather(ci + NBUF, b)
            return carry

        lax.fori_loop(1, NPAIR - 1, steady, 0, unroll=False)

        for b in range(NBUF):
            ci = (NPAIR - 1) * NBUF + b
            drain_gather(b)
            drain_out(b)
            compute(b)
            issue_out(ci, b)
        for b in range(NBUF):
            drain_out(b)

    return body(feat_table, nodes_pad, neigh_flat)


@jax.jit
def kernel(feat_table, nodes, neigh_idx):
    return _sc_encoder(feat_table, nodes, neigh_idx.reshape(-1))


# speedup vs baseline: 1.7298x; 1.1898x over previous
import functools

import jax
import jax.numpy as jnp
from jax import lax
from jax.experimental import pallas as pl
from jax.experimental.pallas import tpu as pltpu
from jax.experimental.pallas import tpu_sc as plsc

N_NODES = 100000
D = 128
BATCH = 50000
S = 10
L = 16

NC = 2
NS = 16
NW = NC * NS

B_PER_W = 1600
BP = NW * B_PER_W
C = 32
NCH = B_PER_W // C
NBUF = 2
NPAIR = NCH // NBUF


def _sc_encoder(feat_table, nodes_pad, neigh_flat):
    mesh = plsc.VectorSubcoreMesh(core_axis_name="c", subcore_axis_name="s")

    @functools.partial(
        pl.kernel,
        mesh=mesh,
        out_type=jax.ShapeDtypeStruct((BATCH, 2 * D), jnp.float32),
        scratch_types=[
            pltpu.VMEM((B_PER_W,), jnp.int32),
            pltpu.VMEM((B_PER_W * S,), jnp.int32),
            pltpu.VMEM((NBUF * C * S, D), jnp.float32),
            pltpu.VMEM((NBUF * C, 2 * D), jnp.float32),
            pltpu.SemaphoreType.DMA,
            pltpu.SemaphoreType.DMA,
            pltpu.SemaphoreType.DMA,
            pltpu.SemaphoreType.DMA,
        ],
    )
    def body(table_h, nodes_h, neigh_h, out_h, sidx, nidx, nrows, obuf,
             gsem0, gsem1, osem0, osem1):
        gsem = (gsem0, gsem1)
        osem = (osem0, osem1)
        wid = lax.axis_index("s") * NC + lax.axis_index("c")
        base = jnp.minimum(wid * B_PER_W, BATCH - B_PER_W)
        pltpu.sync_copy(nodes_h.at[pl.ds(base, B_PER_W)], sidx)
        for j in range(S):
            pltpu.sync_copy(
                neigh_h.at[pl.ds(j * BATCH + base, B_PER_W)],
                nidx.at[pl.ds(j * B_PER_W, B_PER_W)],
            )

        def issue_gather(ci, b):
            off = ci * C
            pltpu.async_copy(
                table_h.at[sidx.at[pl.ds(off, C)]],
                obuf.at[pl.ds(b * C, C), pl.ds(0, D)],
                gsem[b],
            )
            for j in range(S):
                pltpu.async_copy(
                    table_h.at[nidx.at[pl.ds(j * B_PER_W + off, C)]],
                    nrows.at[pl.ds(b * C * S + j * C, C)],
                    gsem[b],
                )

        def drain_gather(b):
            pltpu.make_async_copy(
                table_h.at[pl.ds(0, C)],
                obuf.at[pl.ds(b * C, C), pl.ds(0, D)],
                gsem[b],
            ).wait()
            pltpu.make_async_copy(
                table_h.at[pl.ds(0, C * S)],
                nrows.at[pl.ds(b * C * S, C * S)],
                gsem[b],
            ).wait()

        def compute(b):
            nb = b * C * S
            ob = b * C

            @plsc.parallel_loop(0, C, step=1, unroll=4)
            def row(r):
                rb = nb + r
                for g in range(D // L):
                    sl = pl.ds(g * L, L)
                    acc = nrows[rb, sl]
                    for j in range(1, S):
                        acc = acc + nrows[rb + j * C, sl]
                    obuf[ob + r, pl.ds(D + g * L, L)] = acc * jnp.float32(1.0 / S)

        def issue_out(ci, b):
            pltpu.async_copy(
                obuf.at[pl.ds(b * C, C)],
                out_h.at[pl.ds(base + ci * C, C)],
                osem[b],
            )

        def drain_out(b):
            pltpu.make_async_copy(
                obuf.at[pl.ds(b * C, C)],
                out_h.at[pl.ds(0, C)],
                osem[b],
            ).wait()

        for b in range(NBUF):
            issue_gather(b, b)
        for b in range(NBUF):
            drain_gather(b)
            compute(b)
            issue_out(b, b)
            issue_gather(b + NBUF, b)

        def steady(gi, carry):
            for b in range(NBUF):
                ci = gi * NBUF + b
                drain_gather(b)
                drain_out(b)
                compute(b)
                issue_out(ci, b)
                issue_gather(ci + NBUF, b)
            return carry

        lax.fori_loop(1, NPAIR - 1, steady, 0, unroll=False)

        for b in range(NBUF):
            ci = (NPAIR - 1) * NBUF + b
            drain_gather(b)
            drain_out(b)
            compute(b)
            issue_out(ci, b)
        for b in range(NBUF):
            drain_out(b)

    return body(feat_table, nodes_pad, neigh_flat)


@jax.jit
def kernel(feat_table, nodes, neigh_idx):
    return _sc_encoder(feat_table, nodes, neigh_idx.T.reshape(-1))


# speedup vs baseline: 1.7967x; 1.0387x over previous
import functools

import jax
import jax.numpy as jnp
from jax import lax
from jax.experimental import pallas as pl
from jax.experimental.pallas import tpu as pltpu
from jax.experimental.pallas import tpu_sc as plsc

N_NODES = 100000
D = 128
BATCH = 50000
S = 10
L = 16

NC = 2
NS = 16
NW = NC * NS

B_PER_W = 1600
BP = NW * B_PER_W
C = 32
NCH = B_PER_W // C
NBUF = 2
NPAIR = NCH // NBUF


def _sc_encoder(feat_table, nodes_pad, neigh_flat):
    mesh = plsc.VectorSubcoreMesh(core_axis_name="c", subcore_axis_name="s")

    @functools.partial(
        pl.kernel,
        mesh=mesh,
        out_type=jax.ShapeDtypeStruct((BATCH, 2 * D), jnp.float32),
        scratch_types=[
            pltpu.VMEM((B_PER_W,), jnp.int32),
            pltpu.VMEM((B_PER_W * S,), jnp.int32),
            pltpu.VMEM((NBUF * C * S, D), jnp.float32),
            pltpu.VMEM((NBUF * C, 2 * D), jnp.float32),
            pltpu.SemaphoreType.DMA,
            pltpu.SemaphoreType.DMA,
            pltpu.SemaphoreType.DMA,
            pltpu.SemaphoreType.DMA,
        ],
    )
    def body(table_h, nodes_h, neigh_h, out_h, sidx, nidx, nrows, obuf,
             gsem0, gsem1, osem0, osem1):
        gsem = (gsem0, gsem1)
        osem = (osem0, osem1)
        wid = lax.axis_index("s") * NC + lax.axis_index("c")
        base = jnp.minimum(wid * B_PER_W, BATCH - B_PER_W)
        pltpu.async_copy(nodes_h.at[pl.ds(base, B_PER_W)], sidx, osem[0])
        for j in range(S):
            pltpu.async_copy(
                neigh_h.at[pl.ds(j * BATCH + base, B_PER_W)],
                nidx.at[pl.ds(j * B_PER_W, B_PER_W)],
                osem[0],
            )
        pltpu.make_async_copy(nodes_h.at[pl.ds(base, B_PER_W)], sidx, osem[0]).wait()
        pltpu.make_async_copy(
            neigh_h.at[pl.ds(0, B_PER_W * S)], nidx, osem[0]
        ).wait()

        def issue_gather(ci, b):
            off = ci * C
            pltpu.async_copy(
                table_h.at[sidx.at[pl.ds(off, C)]],
                obuf.at[pl.ds(b * C, C), pl.ds(0, D)],
                gsem[b],
            )
            for j in range(S):
                pltpu.async_copy(
                    table_h.at[nidx.at[pl.ds(j * B_PER_W + off, C)]],
                    nrows.at[pl.ds(b * C * S + j * C, C)],
                    gsem[b],
                )

        def drain_gather(b):
            pltpu.make_async_copy(
                table_h.at[pl.ds(0, C * (S + 1))],
                nrows.at[pl.ds(0, C * (S + 1))],
                gsem[b],
            ).wait()

        def compute(b):
            nb = b * C * S
            ob = b * C

            @plsc.parallel_loop(0, C, step=1, unroll=4)
            def row(r):
                rb = nb + r
                for g in range(D // L):
                    sl = pl.ds(g * L, L)
                    acc = nrows[rb, sl]
                    for j in range(1, S):
                        acc = acc + nrows[rb + j * C, sl]
                    obuf[ob + r, pl.ds(D + g * L, L)] = acc * jnp.float32(1.0 / S)

        def issue_out(ci, b):
            pltpu.async_copy(
                obuf.at[pl.ds(b * C, C)],
                out_h.at[pl.ds(base + ci * C, C)],
                osem[b],
            )

        def drain_out(b):
            pltpu.make_async_copy(
                obuf.at[pl.ds(b * C, C)],
                out_h.at[pl.ds(0, C)],
                osem[b],
            ).wait()

        for b in range(NBUF):
            issue_gather(b, b)
        for b in range(NBUF):
            drain_gather(b)
            compute(b)
            issue_out(b, b)
            issue_gather(b + NBUF, b)

        def steady(gi, carry):
            for b in range(NBUF):
                ci = gi * NBUF + b
                drain_gather(b)
                drain_out(b)
                compute(b)
                issue_out(ci, b)
                issue_gather(ci + NBUF, b)
            return carry

        lax.fori_loop(1, NPAIR - 1, steady, 0, unroll=False)

        for b in range(NBUF):
            ci = (NPAIR - 1) * NBUF + b
            drain_gather(b)
            drain_out(b)
            compute(b)
            issue_out(ci, b)
        for b in range(NBUF):
            drain_out(b)

    return body(feat_table, nodes_pad, neigh_flat)


@jax.jit
def kernel(feat_table, nodes, neigh_idx):
    return _sc_encoder(feat_table, nodes, neigh_idx.T.reshape(-1))
